# Initial kernel scaffold; baseline (speedup 1.0000x reference)
import jax
import jax.numpy as jnp
from jax.experimental import pallas as pl


def kernel(node_feats, edge_index, graph_ids, W0, b0, Wr0, br0, gamma0, beta0, W1, b1, Wr1, br1, gamma1, beta1, Wa, ba):
    raise NotImplementedError("write your pallas kernel here")


# speedup vs baseline: 3.7160x; 3.7160x over previous
import functools
import math

import jax
import jax.numpy as jnp
from jax import lax
from jax.experimental import pallas as pl
from jax.experimental.pallas import tpu as pltpu
from jax.experimental.pallas import tpu_sc as plsc

N = 10000
E = 320000
D_IN = 128
H = 256
G = 256
NC = 2
NS = 16
NW = NC * NS

ECHUNK = 80
IOTILES = 10
IOROWS = N // IOTILES

_BN_SCALE = float(1.0 / math.sqrt(1.0 + 1e-5))

_mesh = plsc.VectorSubcoreMesh(core_axis_name="c", subcore_axis_name="s")


def _make_sc_agg(feature_split: bool):
  epw = E // NS if feature_split else E // NW
  nchunks = epw // ECHUNK
  assert nchunks * ECHUNK == epw

  @functools.partial(
      pl.kernel,
      out_type=jax.ShapeDtypeStruct((2 * N, D_IN), jnp.float32),
      mesh=_mesh,
      scratch_types=[
          pltpu.VMEM((ECHUNK,), jnp.int32),
          pltpu.VMEM((ECHUNK,), jnp.int32),
          pltpu.VMEM((ECHUNK, D_IN), jnp.float32),
          pltpu.VMEM_SHARED((N, D_IN), jnp.float32),
          pltpu.SemaphoreType.DMA,
      ],
  )
  def kern(table_hbm, src_hbm, dst_hbm, zeros_hbm, out_hbm,
           src_v, dst_v, rows_v, acc_sh, sem):
    c = lax.axis_index("c")
    s = lax.axis_index("s")

    @pl.when(s < IOTILES)
    def _():
      pltpu.sync_copy(zeros_hbm.at[pl.ds(s * IOROWS, IOROWS)],
                      acc_sh.at[pl.ds(s * IOROWS, IOROWS)])
    plsc.subcore_barrier()

    if feature_split:
      ebase = s * epw
      toff = c * N
    else:
      ebase = (c * NS + s) * epw
      toff = 0

    def body(i, carry):
      base = ebase + i * ECHUNK
      pltpu.sync_copy(src_hbm.at[pl.ds(base, ECHUNK)], src_v)
      pltpu.sync_copy(dst_hbm.at[pl.ds(base, ECHUNK)], dst_v)
      if feature_split:
        off = jnp.full((16,), toff, jnp.int32)
        for t in range(ECHUNK // 16):
          sl = src_v[pl.ds(t * 16, 16)]
          src_v[pl.ds(t * 16, 16)] = sl + off
      pltpu.async_copy(table_hbm.at[src_v], rows_v, sem).wait()
      pltpu.sync_copy(rows_v, acc_sh.at[dst_v], add=True)
      return carry

    lax.fori_loop(0, nchunks, body, 0)
    plsc.subcore_barrier()

    @pl.when(s < IOTILES)
    def _():
      pltpu.sync_copy(acc_sh.at[pl.ds(s * IOROWS, IOROWS)],
                      out_hbm.at[pl.ds(c * N + s * IOROWS, IOROWS)])

  return kern


_sc_agg_edge_split = _make_sc_agg(feature_split=False)
_sc_agg_feat_split = _make_sc_agg(feature_split=True)


NP = 10240
RPW = NP // NW
RCHUNK = 64
ACC_WORDS = G * H


@functools.partial(
    pl.kernel,
    out_type=jax.ShapeDtypeStruct((NW * ACC_WORDS,), jnp.float32),
    mesh=_mesh,
    compiler_params=pltpu.CompilerParams(needs_layout_passes=False),
    scratch_types=[
        pltpu.VMEM((RPW,), jnp.int32),
        pltpu.VMEM((RCHUNK * H,), jnp.float32),
        pltpu.VMEM((ACC_WORDS,), jnp.float32),
    ],
)
def _sc_segmax(h2_hbm, ids_hbm, out_hbm, ids_v, rows_v, acc_v):
  c = lax.axis_index("c")
  s = lax.axis_index("s")
  wid = c * NS + s
  neg_inf = jnp.full((16,), -jnp.inf, jnp.float32)

  def init_body(k, carry):
    acc_v[pl.ds(k * 16, 16)] = neg_inf
    return carry
  lax.fori_loop(0, ACC_WORDS // 16, init_body, 0)

  pltpu.sync_copy(ids_hbm.at[pl.ds(wid * RPW, RPW)], ids_v)
  lane = lax.iota(jnp.int32, 16)

  def chunk_body(cc, carry):
    pltpu.sync_copy(
        h2_hbm.at[pl.ds((wid * RPW + cc * RCHUNK) * H, RCHUNK * H)], rows_v)

    def row_body(rl, carry2):
      rglob = cc * RCHUNK + rl
      gid = plsc.load_gather(ids_v, [jnp.full((16,), rglob, jnp.int32)])
      base = gid * H
      for j in range(H // 16):
        addr = base + (lane + j * 16)
        cur = plsc.load_gather(acc_v, [addr])
        vals = rows_v[pl.ds(rl * H + j * 16, 16)]
        plsc.store_scatter(acc_v, [addr], jnp.maximum(cur, vals))
      return carry2

    lax.fori_loop(0, RCHUNK, row_body, 0)
    return carry

  lax.fori_loop(0, RPW // RCHUNK, chunk_body, 0)
  pltpu.sync_copy(acc_v, out_hbm.at[pl.ds(wid * ACC_WORDS, ACC_WORDS)])


BR = 1000
NBLK = N // BR


def _tc_layer1_body(p0, p1, nf, w0, b0, wr0, br0, g0, be0, out):
  agg = p0[...] + p1[...]
  x = jnp.maximum(
      jnp.dot(agg, w0[...], preferred_element_type=jnp.float32) + b0[...], 0.0)
  r = jnp.maximum(
      jnp.dot(nf[...], wr0[...], preferred_element_type=jnp.float32) + br0[...],
      0.0)
  h = (x + r) * (g0[...] * _BN_SCALE) + be0[...]
  out[0] = h[:, :D_IN]
  out[1] = h[:, D_IN:]


def _tc_layer2_body(q0, q1, h1a, h1b, w1, b1, wr1, br1, g1, be1, wa, ba, ids,
                    h2_out, ssum_out):
  i = pl.program_id(0)
  w1v = w1[...]
  wr1v = wr1[...]
  agg = (jnp.dot(q0[...], w1v[:D_IN, :], preferred_element_type=jnp.float32) +
         jnp.dot(q1[...], w1v[D_IN:, :], preferred_element_type=jnp.float32))
  res = (jnp.dot(h1a[...], wr1v[:D_IN, :], preferred_element_type=jnp.float32) +
         jnp.dot(h1b[...], wr1v[D_IN:, :], preferred_element_type=jnp.float32))
  x = jnp.maximum(agg + b1[...], 0.0)
  r = jnp.maximum(res + br1[...], 0.0)
  h2 = (x + r) * (g1[...] * _BN_SCALE) + be1[...]
  h2_out[...] = h2
  w = jax.nn.sigmoid(
      jnp.sum(h2 * wa[...], axis=1, keepdims=True) + ba[0, 0])
  wh = w * h2
  idvec = ids[0, 0, :]
  onehot = (idvec[:, None] == lax.broadcasted_iota(jnp.int32, (BR, G), 1)
            ).astype(jnp.float32)
  part = lax.dot_general(onehot, wh, (((0,), (0,)), ((), ())),
                         preferred_element_type=jnp.float32)

  @pl.when(i == 0)
  def _():
    ssum_out[...] = jnp.zeros_like(ssum_out)

  ssum_out[...] += part


def _tc_combine_body(partials, ssum, out):
  out[:, :H] = ssum[...]
  out[:, H:] = jnp.max(partials[...], axis=0)


def kernel(node_feats, edge_index, graph_ids, W0, b0, Wr0, br0, gamma0, beta0,
           W1, b1, Wr1, br1, gamma1, beta1, Wa, ba):
  src = edge_index[0]
  dst = edge_index[1]
  zeros = jnp.zeros((N, D_IN), jnp.float32)

  p = _sc_agg_edge_split(node_feats, src, dst, zeros)

  row_blk = lambda i: (i, 0)
  vec_blk = lambda i: (0, 0)
  h1 = pl.pallas_call(
      _tc_layer1_body,
      grid=(NBLK,),
      in_specs=[
          pl.BlockSpec((BR, D_IN), row_blk),
          pl.BlockSpec((BR, D_IN), lambda i: (i + NBLK, 0)),
          pl.BlockSpec((BR, D_IN), row_blk),
          pl.BlockSpec((D_IN, H), vec_blk),
          pl.BlockSpec((1, H), vec_blk),
          pl.BlockSpec((D_IN, H), vec_blk),
          pl.BlockSpec((1, H), vec_blk),
          pl.BlockSpec((1, H), vec_blk),
          pl.BlockSpec((1, H), vec_blk),
      ],
      out_specs=pl.BlockSpec((2, BR, D_IN), lambda i: (0, i, 0)),
      out_shape=jax.ShapeDtypeStruct((2, N, D_IN), jnp.float32),
  )(p, p, node_feats, W0, b0.reshape(1, H), Wr0, br0.reshape(1, H),
    gamma0.reshape(1, H), beta0.reshape(1, H))
  h1flat = h1.reshape(2 * N, D_IN)

  q = _sc_agg_feat_split(h1flat, src, dst, zeros)

  ids3 = graph_ids.reshape(NBLK, 1, BR)
  h2, ssum = pl.pallas_call(
      _tc_layer2_body,
      grid=(NBLK,),
      in_specs=[
          pl.BlockSpec((BR, D_IN), row_blk),
          pl.BlockSpec((BR, D_IN), lambda i: (i + NBLK, 0)),
          pl.BlockSpec((BR, D_IN), row_blk),
          pl.BlockSpec((BR, D_IN), lambda i: (i + NBLK, 0)),
          pl.BlockSpec((H, H), vec_blk),
          pl.BlockSpec((1, H), vec_blk),
          pl.BlockSpec((H, H), vec_blk),
          pl.BlockSpec((1, H), vec_blk),
          pl.BlockSpec((1, H), vec_blk),
          pl.BlockSpec((1, H), vec_blk),
          pl.BlockSpec((1, H), vec_blk),
          pl.BlockSpec((1, 1), vec_blk),
          pl.BlockSpec((1, 1, BR), lambda i: (i, 0, 0)),
      ],
      out_specs=[
          pl.BlockSpec((BR, H), row_blk),
          pl.BlockSpec((G, H), vec_blk),
      ],
      out_shape=[
          jax.ShapeDtypeStruct((N, H), jnp.float32),
          jax.ShapeDtypeStruct((G, H), jnp.float32),
      ],
  )(q, q, h1flat, h1flat, W1, b1.reshape(1, H), Wr1, br1.reshape(1, H),
    gamma1.reshape(1, H), beta1.reshape(1, H), Wa.reshape(1, H),
    ba.reshape(1, 1), ids3)

  h2p = jnp.pad(h2, ((0, NP - N), (0, 0)), constant_values=-jnp.inf)
  idsp = jnp.pad(graph_ids, (0, NP - N))
  partials = _sc_segmax(h2p.reshape(NP * H), idsp)

  out = pl.pallas_call(
      _tc_combine_body,
      grid=(1,),
      in_specs=[
          pl.BlockSpec((NW, G, H), lambda i: (0, 0, 0)),
          pl.BlockSpec((G, H), lambda i: (0, 0)),
      ],
      out_specs=pl.BlockSpec((G, 2 * H), lambda i: (0, 0)),
      out_shape=jax.ShapeDtypeStruct((G, 2 * H), jnp.float32),
  )(partials.reshape(NW, G, H), ssum)
  return out


# speedup vs baseline: 8.2455x; 2.2189x over previous
import functools
import math

import jax
import jax.numpy as jnp
from jax import lax
from jax.experimental import pallas as pl
from jax.experimental.pallas import tpu as pltpu
from jax.experimental.pallas import tpu_sc as plsc

N = 10000
E = 320000
D_IN = 128
H = 256
G = 256
NC = 2
NS = 16
NW = NC * NS

ECHUNK = 128
EPW = 10240
EPAD = EPW * NW
ROWS = EPAD // ECHUNK
TPAD = 64
IOTILES = 10
IOROWS = N // IOTILES

_BN_SCALE = float(1.0 / math.sqrt(1.0 + 1e-5))

_mesh = plsc.VectorSubcoreMesh(core_axis_name="c", subcore_axis_name="s")


def _make_sc_agg(feature_split: bool):
  epw = EPAD // NS if feature_split else EPW
  ncpt = epw // ECHUNK

  @functools.partial(
      pl.kernel,
      out_type=jax.ShapeDtypeStruct((2 * N, D_IN), jnp.float32),
      mesh=_mesh,
      scratch_types=[
          pltpu.VMEM((ECHUNK,), jnp.int32),
          pltpu.VMEM((ECHUNK,), jnp.int32),
          pltpu.VMEM((ECHUNK,), jnp.int32),
          pltpu.VMEM((ECHUNK,), jnp.int32),
          pltpu.VMEM((ECHUNK, D_IN), jnp.float32),
          pltpu.VMEM((ECHUNK, D_IN), jnp.float32),
          pltpu.VMEM_SHARED((N, D_IN), jnp.float32),
          pltpu.SemaphoreType.DMA,
          pltpu.SemaphoreType.DMA,
          pltpu.SemaphoreType.DMA,
          pltpu.SemaphoreType.DMA,
          pltpu.SemaphoreType.DMA,
          pltpu.SemaphoreType.DMA,
      ],
  )
  def kern(table_hbm, srcf_hbm, dstf_hbm, zeros_hbm, out_hbm,
           src0_v, src1_v, dst0_v, dst1_v, rows0_v, rows1_v, acc_sh,
           gsem0, gsem1, ssem0, ssem1, dsem0, dsem1):
    c = lax.axis_index("c")
    s = lax.axis_index("s")
    bufs = ((rows0_v, src0_v, dst0_v, gsem0, ssem0, dsem0),
            (rows1_v, src1_v, dst1_v, gsem1, ssem1, dsem1))

    @pl.when(s < IOTILES)
    def _():
      pltpu.sync_copy(zeros_hbm.at[pl.ds(s * IOROWS, IOROWS)],
                      acc_sh.at[pl.ds(s * IOROWS, IOROWS)])

    if feature_split:
      ebase = c * EPAD + s * epw
      dbase = s * epw
    else:
      ebase = (c * NS + s) * epw
      dbase = ebase
    plsc.subcore_barrier()

    def istart_src(b, j):
      _, sv, _, _, ss, _ = bufs[b]
      pltpu.make_async_copy(
          srcf_hbm.at[pl.ds(ebase + j * ECHUNK, ECHUNK)], sv, ss).start()

    def iwait_src(b):
      _, sv, _, _, ss, _ = bufs[b]
      pltpu.make_async_copy(srcf_hbm.at[pl.ds(0, ECHUNK)], sv, ss).wait()

    def istart_dst(b, j):
      _, _, dv, _, _, ds_ = bufs[b]
      pltpu.make_async_copy(
          dstf_hbm.at[pl.ds(dbase + j * ECHUNK, ECHUNK)], dv, ds_).start()

    def iwait_dst(b):
      _, _, dv, _, _, ds_ = bufs[b]
      pltpu.make_async_copy(dstf_hbm.at[pl.ds(0, ECHUNK)], dv, ds_).wait()

    def gstart(b):
      buf, sv, _, gs, _, _ = bufs[b]
      pltpu.make_async_copy(table_hbm.at[sv], buf, gs).start()

    def gwait(b):
      buf, sv, _, gs, _, _ = bufs[b]
      pltpu.make_async_copy(table_hbm.at[sv], buf, gs).wait()

    def scat(b):
      buf, _, dv, _, _, _ = bufs[b]
      pltpu.sync_copy(buf, acc_sh.at[dv], add=True)

    for b in range(2):
      istart_src(b, b)
      istart_dst(b, b)
    for b in range(2):
      iwait_src(b)
      gstart(b)

    def body(i, carry):
      j0 = 2 * i
      for b in range(2):
        j = j0 + b
        gwait(b)

        @pl.when(j + 2 < ncpt)
        def _():
          istart_src(b, j + 2)

        iwait_dst(b)
        scat(b)

        @pl.when(j + 2 < ncpt)
        def _():
          istart_dst(b, j + 2)
          iwait_src(b)
          gstart(b)
      return carry

    lax.fori_loop(0, ncpt // 2, body, 0)
    plsc.subcore_barrier()

    @pl.when(s < IOTILES)
    def _():
      pltpu.sync_copy(acc_sh.at[pl.ds(s * IOROWS, IOROWS)],
                      out_hbm.at[pl.ds(c * N + s * IOROWS, IOROWS)])

  return kern


_sc_agg_edge_split = _make_sc_agg(feature_split=False)
_sc_agg_feat_split = _make_sc_agg(feature_split=True)


NP = 10240
RPW = NP // NW
RCHUNK = 64
ACC_WORDS = G * H


@functools.partial(
    pl.kernel,
    out_type=jax.ShapeDtypeStruct((NW * ACC_WORDS,), jnp.float32),
    mesh=_mesh,
    compiler_params=pltpu.CompilerParams(needs_layout_passes=False),
    scratch_types=[
        pltpu.VMEM((RPW,), jnp.int32),
        pltpu.VMEM((RCHUNK * H,), jnp.float32),
        pltpu.VMEM((ACC_WORDS,), jnp.float32),
    ],
)
def _sc_segmax(h2_hbm, ids_hbm, out_hbm, ids_v, rows_v, acc_v):
  c = lax.axis_index("c")
  s = lax.axis_index("s")
  wid = c * NS + s
  neg_inf = jnp.full((16,), -jnp.inf, jnp.float32)

  def init_body(k, carry):
    acc_v[pl.ds(k * 16, 16)] = neg_inf
    return carry
  lax.fori_loop(0, ACC_WORDS // 16, init_body, 0)

  pltpu.sync_copy(ids_hbm.at[pl.ds(wid * RPW, RPW)], ids_v)
  lane = lax.iota(jnp.int32, 16)

  def chunk_body(cc, carry):
    pltpu.sync_copy(
        h2_hbm.at[pl.ds((wid * RPW + cc * RCHUNK) * H, RCHUNK * H)], rows_v)

    def row_body(rl, carry2):
      rglob = cc * RCHUNK + rl
      gid = plsc.load_gather(ids_v, [jnp.full((16,), rglob, jnp.int32)])
      base = gid * H
      for j in range(H // 16):
        addr = base + (lane + j * 16)
        cur = plsc.load_gather(acc_v, [addr])
        vals = rows_v[pl.ds(rl * H + j * 16, 16)]
        plsc.store_scatter(acc_v, [addr], jnp.maximum(cur, vals))
      return carry2

    lax.fori_loop(0, RCHUNK, row_body, 0)
    return carry

  lax.fori_loop(0, RPW // RCHUNK, chunk_body, 0)
  pltpu.sync_copy(acc_v, out_hbm.at[pl.ds(wid * ACC_WORDS, ACC_WORDS)])


BR = 1000
NBLK = N // BR


def _tc_layer1_body(p0, p1, nf, w0, b0, wr0, br0, g0, be0, out):
  agg = p0[...] + p1[...]
  x = jnp.maximum(
      jnp.dot(agg, w0[...], preferred_element_type=jnp.float32) + b0[...], 0.0)
  r = jnp.maximum(
      jnp.dot(nf[...], wr0[...], preferred_element_type=jnp.float32) + br0[...],
      0.0)
  h = (x + r) * (g0[...] * _BN_SCALE) + be0[...]
  out[0] = h[:, :D_IN]
  out[1] = h[:, D_IN:]


def _tc_layer2_body(q0, q1, h1a, h1b, w1, b1, wr1, br1, g1, be1, wa, ba, ids,
                    h2_out, ssum_out):
  i = pl.program_id(0)
  agg = jnp.dot(jnp.concatenate([q0[...], q1[...]], axis=1), w1[...],
                preferred_element_type=jnp.float32)
  res = jnp.dot(jnp.concatenate([h1a[...], h1b[...]], axis=1), wr1[...],
                preferred_element_type=jnp.float32)
  x = jnp.maximum(agg + b1[...], 0.0)
  r = jnp.maximum(res + br1[...], 0.0)
  h2 = (x + r) * (g1[...] * _BN_SCALE) + be1[...]
  h2_out[...] = h2
  w = jax.nn.sigmoid(
      jnp.dot(h2, wa[...], preferred_element_type=jnp.float32) + ba[0, 0])
  wh = w * h2
  idvec = ids[0, 0, :]
  onehot = (idvec[:, None] == lax.broadcasted_iota(jnp.int32, (BR, G), 1)
            ).astype(jnp.float32)
  part = lax.dot_general(onehot, wh, (((0,), (0,)), ((), ())),
                         preferred_element_type=jnp.float32, precision=lax.Precision.HIGHEST)

  @pl.when(i == 0)
  def _():
    ssum_out[...] = jnp.zeros_like(ssum_out)

  ssum_out[...] += part


def _tc_combine_body(partials, ssum, out):
  out[:, :H] = ssum[...]
  out[:, H:] = jnp.max(partials[...], axis=0)


def kernel(node_feats, edge_index, graph_ids, W0, b0, Wr0, br0, gamma0, beta0,
           W1, b1, Wr1, br1, gamma1, beta1, Wa, ba):
  src = edge_index[0]
  dst = edge_index[1]
  zeros = jnp.zeros((N, D_IN), jnp.float32)

  ar = jnp.arange(EPAD - E, dtype=jnp.int32)
  srcf = jnp.concatenate([src, N + ar % TPAD])
  dstf = jnp.concatenate([dst, ar % N])
  pad2 = 2 * N + ar % TPAD
  srcf2 = jnp.concatenate([src, pad2, src + N, pad2])
  nf_pad = jnp.pad(node_feats, ((0, TPAD), (0, 0)))

  p = _sc_agg_edge_split(nf_pad, srcf, dstf, zeros)

  row_blk = lambda i: (i, 0)
  vec_blk = lambda i: (0, 0)
  h1 = pl.pallas_call(
      _tc_layer1_body,
      grid=(NBLK,),
      in_specs=[
          pl.BlockSpec((BR, D_IN), row_blk),
          pl.BlockSpec((BR, D_IN), lambda i: (i + NBLK, 0)),
          pl.BlockSpec((BR, D_IN), row_blk),
          pl.BlockSpec((D_IN, H), vec_blk),
          pl.BlockSpec((1, H), vec_blk),
          pl.BlockSpec((D_IN, H), vec_blk),
          pl.BlockSpec((1, H), vec_blk),
          pl.BlockSpec((1, H), vec_blk),
          pl.BlockSpec((1, H), vec_blk),
      ],
      out_specs=pl.BlockSpec((2, BR, D_IN), lambda i: (0, i, 0)),
      out_shape=jax.ShapeDtypeStruct((2, N, D_IN), jnp.float32),
  )(p, p, node_feats, W0, b0.reshape(1, H), Wr0, br0.reshape(1, H),
    gamma0.reshape(1, H), beta0.reshape(1, H))
  h1flat = h1.reshape(2 * N, D_IN)

  q = _sc_agg_feat_split(jnp.pad(h1flat, ((0, TPAD), (0, 0))), srcf2, dstf,
                         zeros)

  ids3 = graph_ids.reshape(NBLK, 1, BR)
  h2, ssum = pl.pallas_call(
      _tc_layer2_body,
      grid=(NBLK,),
      in_specs=[
          pl.BlockSpec((BR, D_IN), row_blk),
          pl.BlockSpec((BR, D_IN), lambda i: (i + NBLK, 0)),
          pl.BlockSpec((BR, D_IN), row_blk),
          pl.BlockSpec((BR, D_IN), lambda i: (i + NBLK, 0)),
          pl.BlockSpec((H, H), vec_blk),
          pl.BlockSpec((1, H), vec_blk),
          pl.BlockSpec((H, H), vec_blk),
          pl.BlockSpec((1, H), vec_blk),
          pl.BlockSpec((1, H), vec_blk),
          pl.BlockSpec((1, H), vec_blk),
          pl.BlockSpec((H, 1), vec_blk),
          pl.BlockSpec((1, 1), vec_blk),
          pl.BlockSpec((1, 1, BR), lambda i: (i, 0, 0)),
      ],
      out_specs=[
          pl.BlockSpec((BR, H), row_blk),
          pl.BlockSpec((G, H), vec_blk),
      ],
      out_shape=[
          jax.ShapeDtypeStruct((N, H), jnp.float32),
          jax.ShapeDtypeStruct((G, H), jnp.float32),
      ],
  )(q, q, h1flat, h1flat, W1, b1.reshape(1, H), Wr1, br1.reshape(1, H),
    gamma1.reshape(1, H), beta1.reshape(1, H), Wa,
    ba.reshape(1, 1), ids3)

  h2p = jnp.pad(h2, ((0, NP - N), (0, 0)), constant_values=-jnp.inf)
  idsp = jnp.pad(graph_ids, (0, NP - N))
  partials = _sc_segmax(h2p.reshape(NP * H), idsp)

  out = pl.pallas_call(
      _tc_combine_body,
      grid=(1,),
      in_specs=[
          pl.BlockSpec((NW, G, H), lambda i: (0, 0, 0)),
          pl.BlockSpec((G, H), lambda i: (0, 0)),
      ],
      out_specs=pl.BlockSpec((G, 2 * H), lambda i: (0, 0)),
      out_shape=jax.ShapeDtypeStruct((G, 2 * H), jnp.float32),
  )(partials.reshape(NW, G, H), ssum)
  return out
